# baseline (device time: 22923 ns/iter reference)
import jax
import jax.numpy as jnp
from jax import lax
from jax.experimental import pallas as pl
from jax.experimental.pallas import tpu as pltpu

N_CHUNKS = 8


def kernel(x):
    _, m, n = x.shape
    mc = m // N_CHUNKS

    def body(x_hbm, out_hbm, xf32, xsend, xrecv, ysend, yrecv, oown, ooth,
             in_sems, oown_sems, ooth_sems,
             x_send_sems, x_recv_sems, y_send_sems, y_recv_sems):
        my_x = lax.axis_index("x")
        my_y = lax.axis_index("y")
        x_partner = (1 - my_x, my_y)
        y_partner = (my_x, 1 - my_y)
        own_col = pl.ds(my_y * n, n)
        oth_col = pl.ds((1 - my_y) * n, n)

        def rows(c):
            return pl.ds(c * mc, mc)

        in_dmas = []
        for c in range(N_CHUNKS):
            dma = pltpu.make_async_copy(
                x_hbm.at[0, rows(c), :], xf32.at[rows(c)], in_sems.at[c])
            dma.start()
            in_dmas.append(dma)

        barrier = pltpu.get_barrier_semaphore()
        for nbr in (x_partner, y_partner):
            pl.semaphore_signal(barrier, inc=1, device_id=nbr,
                                device_id_type=pl.DeviceIdType.MESH)
        pl.semaphore_wait(barrier, 2)

        x_rdmas = []
        for c in range(N_CHUNKS):
            in_dmas[c].wait()
            xsend[rows(c)] = xf32[rows(c)].astype(jnp.bfloat16)
            rdma = pltpu.make_async_remote_copy(
                src_ref=xsend.at[rows(c)], dst_ref=xrecv.at[rows(c)],
                send_sem=x_send_sems.at[c], recv_sem=x_recv_sems.at[c],
                device_id=x_partner, device_id_type=pl.DeviceIdType.MESH)
            rdma.start()
            x_rdmas.append(rdma)

        y_rdmas = []
        for c in range(N_CHUNKS):
            x_rdmas[c].wait_recv()
            ysend[rows(c)] = xsend[rows(c)] + xrecv[rows(c)]
            rdma = pltpu.make_async_remote_copy(
                src_ref=ysend.at[rows(c)], dst_ref=yrecv.at[rows(c)],
                send_sem=y_send_sems.at[c], recv_sem=y_recv_sems.at[c],
                device_id=y_partner, device_id_type=pl.DeviceIdType.MESH)
            rdma.start()
            y_rdmas.append(rdma)

        oown_dmas = []
        for c in range(N_CHUNKS):
            oown[rows(c)] = ysend[rows(c)].astype(jnp.float32)
            dma = pltpu.make_async_copy(
                oown.at[rows(c)], out_hbm.at[rows(c), own_col],
                oown_sems.at[c])
            dma.start()
            oown_dmas.append(dma)

        ooth_dmas = []
        for c in range(N_CHUNKS):
            y_rdmas[c].wait_recv()
            ooth[rows(c)] = yrecv[rows(c)].astype(jnp.float32)
            dma = pltpu.make_async_copy(
                ooth.at[rows(c)], out_hbm.at[rows(c), oth_col],
                ooth_sems.at[c])
            dma.start()
            ooth_dmas.append(dma)

        for c in range(N_CHUNKS):
            oown_dmas[c].wait()
            ooth_dmas[c].wait()
            x_rdmas[c].wait_send()
            y_rdmas[c].wait_send()

    return pl.pallas_call(
        body,
        out_shape=jax.ShapeDtypeStruct((m, 2 * n), jnp.float32),
        in_specs=[pl.BlockSpec(memory_space=pl.ANY)],
        out_specs=pl.BlockSpec(memory_space=pl.ANY),
        scratch_shapes=[
            pltpu.VMEM((m, n), jnp.float32),
            pltpu.VMEM((m, n), jnp.bfloat16),
            pltpu.VMEM((m, n), jnp.bfloat16),
            pltpu.VMEM((m, n), jnp.bfloat16),
            pltpu.VMEM((m, n), jnp.bfloat16),
            pltpu.VMEM((m, n), jnp.float32),
            pltpu.VMEM((m, n), jnp.float32),
            pltpu.SemaphoreType.DMA((N_CHUNKS,)),
            pltpu.SemaphoreType.DMA((N_CHUNKS,)),
            pltpu.SemaphoreType.DMA((N_CHUNKS,)),
            pltpu.SemaphoreType.DMA((N_CHUNKS,)),
            pltpu.SemaphoreType.DMA((N_CHUNKS,)),
            pltpu.SemaphoreType.DMA((N_CHUNKS,)),
            pltpu.SemaphoreType.DMA((N_CHUNKS,)),
        ],
        compiler_params=pltpu.CompilerParams(collective_id=0),
    )(x)
